# Initial kernel scaffold; baseline (speedup 1.0000x reference)
import jax
import jax.numpy as jnp
from jax.experimental import pallas as pl


def kernel(vert_ids, image_array):
    raise NotImplementedError("write your pallas kernel here")


# speedup vs baseline: 3.4900x; 3.4900x over previous
import functools

import jax
import jax.numpy as jnp
from jax import lax
from jax.experimental import pallas as pl
from jax.experimental.pallas import tpu as pltpu
from jax.experimental.pallas import tpu_sc as plsc

H, W, C = 720, 1280, 512
NV = 2562
HW = H * W
KCH = 3
NW = 32
PPT = HW // NW
VPT = PPT // 16
TBL = 4096
VPW = TBL // NW

_MESH = plsc.VectorSubcoreMesh(core_axis_name="c", subcore_axis_name="s")
_PARAMS = pltpu.CompilerParams(needs_layout_passes=False)


@functools.partial(
    pl.kernel,
    mesh=_MESH,
    compiler_params=_PARAMS,
    out_type=jax.ShapeDtypeStruct((NW * TBL,), jnp.int32),
    scratch_types=[
        pltpu.VMEM((PPT * KCH,), jnp.int32),
        pltpu.VMEM((TBL,), jnp.int32),
        pltpu.VMEM((16,), jnp.int32),
    ],
)
def _winner_keys(ids_hbm, tbl_hbm, chunk, tbl, tmp):
    w = lax.axis_index("s") * 2 + lax.axis_index("c")
    pltpu.sync_copy(ids_hbm.at[pl.ds(w * PPT * KCH, PPT * KCH)], chunk)

    lane = lax.iota(jnp.int32, 16)
    shift_up = jnp.minimum(lane + 1, 15)
    is_top = lane == 15

    def init(i, carry):
        tbl[pl.ds(i * 16, 16)] = jnp.full((16,), -1, jnp.int32)
        return carry

    lax.fori_loop(0, TBL // 16, init, 0)

    pix_base = w * PPT
    for k in range(KCH):
        def body(j, carry, k=k):
            ids = plsc.load_gather(chunk, [j * 48 + lane * 3 + k])
            comp = ids * 16 + lane
            scomp, slane = plsc.sort_key_val(comp, lane)
            sid = lax.shift_right_logical(scomp, 4)
            tmp[...] = sid
            nxt = plsc.load_gather(tmp, [shift_up])
            last_of_group = (sid != nxt) | is_top
            key = (k * HW + pix_base + j * 16) + slane
            plsc.store_scatter(tbl, [sid], key, mask=last_of_group)
            return carry

        lax.fori_loop(0, VPT, body, 0)

    pltpu.sync_copy(tbl, tbl_hbm.at[pl.ds(w * TBL, TBL)])


@functools.partial(
    pl.kernel,
    mesh=_MESH,
    compiler_params=_PARAMS,
    out_type=jax.ShapeDtypeStruct((TBL, C), jnp.float32),
    scratch_types=[
        pltpu.VMEM((NW, VPW), jnp.int32),
        pltpu.VMEM((VPW,), jnp.int32),
        pltpu.VMEM((VPW,), jnp.int32),
        pltpu.VMEM((VPW, C), jnp.float32),
        pltpu.SemaphoreType.DMA,
    ],
)
def _gather_rows(tbl_hbm, img_hbm, out_hbm, tb, win, idxv, rows, sem):
    w = lax.axis_index("s") * 2 + lax.axis_index("c")
    vbase = w * VPW
    copies = [
        pltpu.async_copy(tbl_hbm.at[pl.ds(t * TBL + vbase, VPW)], tb.at[t], sem)
        for t in range(NW)
    ]
    for cp in copies:
        cp.wait()

    lane = lax.iota(jnp.int32, 16)
    miss_lanes = jnp.zeros((16,), jnp.int32)
    for v in range(VPW // 16):
        m = tb[0, pl.ds(v * 16, 16)]
        for t in range(1, NW):
            m = jnp.maximum(m, tb[t, pl.ds(v * 16, 16)])
        win[pl.ds(v * 16, 16)] = m
        hit = m >= 0
        vid = vbase + v * 16 + lane
        idxv[pl.ds(v * 16, 16)] = jnp.where(hit, lax.rem(m, HW), vid)
        real_miss = jnp.logical_and(jnp.logical_not(hit), vid < NV)
        miss_lanes = miss_lanes + jnp.where(real_miss, 1, 0)

    pltpu.async_copy(img_hbm.at[idxv], rows, sem).wait()

    n_miss = jnp.max(miss_lanes)

    @pl.when(n_miss > 0)
    def _patch_misses():
        def fix(r, carry):
            rs = jnp.full((16,), 0, jnp.int32) + r
            wk = plsc.load_gather(win, [rs])
            is_miss = wk < 0
            for cb in range(C // 16):
                ci = lane + cb * 16
                seg = plsc.load_gather(rows, [rs, ci])
                plsc.store_scatter(rows, [rs, ci],
                                   jnp.where(is_miss, 0.5, seg))
            return carry

        lax.fori_loop(0, VPW, fix, 0)

    pltpu.sync_copy(rows, out_hbm.at[pl.ds(vbase, VPW)])


def kernel(vert_ids, image_array):
    ids_flat = vert_ids.reshape(HW * KCH)
    img = image_array.reshape(HW, C)
    tbls = _winner_keys(ids_flat)
    padded = _gather_rows(tbls, img)
    return padded[:NV][None]
